# baseline (speedup 1.0000x reference)
import functools

import jax
import jax.numpy as jnp
from jax.experimental import pallas as pl
from jax.experimental.pallas import tpu as pltpu

N_USERS = 20000
N_ITEMS = 30000
N_NODES = N_USERS + N_ITEMS
EMB = 64
ROW_BLK = 2000


def _dense3_body(sv, sa, st, wv, wa, wt, ov, oa, ot):
    ov[...] = jax.nn.leaky_relu(
        jax.lax.dot_general(sv[...], wv[...], (((1,), (1,)), ((), ())),
                            preferred_element_type=jnp.float32), 0.2)
    oa[...] = jax.nn.leaky_relu(
        jax.lax.dot_general(sa[...], wa[...], (((1,), (1,)), ((), ())),
                            preferred_element_type=jnp.float32), 0.2)
    ot[...] = jax.nn.leaky_relu(
        jax.lax.dot_general(st[...], wt[...], (((1,), (1,)), ((), ())),
                            preferred_element_type=jnp.float32), 0.2)


def _dense3_fuse_body(sv, sa, st, wv, wa, wt, uid, out):
    v = jax.nn.leaky_relu(
        jax.lax.dot_general(sv[...], wv[...], (((1,), (1,)), ((), ())),
                            preferred_element_type=jnp.float32), 0.2)
    a = jax.nn.leaky_relu(
        jax.lax.dot_general(sa[...], wa[...], (((1,), (1,)), ((), ())),
                            preferred_element_type=jnp.float32), 0.2)
    t = jax.nn.leaky_relu(
        jax.lax.dot_general(st[...], wt[...], (((1,), (1,)), ((), ())),
                            preferred_element_type=jnp.float32), 0.2)
    out[...] = v + a + t + uid[...]


def _dense3(sv, sa, st, wv, wa, wt):
    n = sv.shape[0]
    grid = n // ROW_BLK
    row_spec = pl.BlockSpec((ROW_BLK, EMB), lambda i: (i, 0))
    w_spec = pl.BlockSpec((EMB, EMB), lambda i: (0, 0))
    return pl.pallas_call(
        _dense3_body,
        grid=(grid,),
        in_specs=[row_spec, row_spec, row_spec, w_spec, w_spec, w_spec],
        out_specs=[row_spec, row_spec, row_spec],
        out_shape=[jax.ShapeDtypeStruct((n, EMB), jnp.float32)] * 3,
    )(sv, sa, st, wv, wa, wt)


def _dense3_fuse(sv, sa, st, wv, wa, wt, uid):
    n = sv.shape[0]
    grid = n // ROW_BLK
    row_spec = pl.BlockSpec((ROW_BLK, EMB), lambda i: (i, 0))
    w_spec = pl.BlockSpec((EMB, EMB), lambda i: (0, 0))
    return pl.pallas_call(
        _dense3_fuse_body,
        grid=(grid,),
        in_specs=[row_spec, row_spec, row_spec, w_spec, w_spec, w_spec, row_spec],
        out_specs=row_spec,
        out_shape=jax.ShapeDtypeStruct((n, EMB), jnp.float32),
    )(sv, sa, st, wv, wa, wt, uid)


def _spmm(adj_indices, adj_values, x):
    gathered = adj_values[:, None] * jnp.take(x, adj_indices[1], axis=0)
    return jax.ops.segment_sum(gathered, adj_indices[0], num_segments=N_NODES)


def kernel(adj_indices, adj_values, user_id_emb, item_id_emb,
           user_visual_emb, user_acoustic_emb, user_textual_emb,
           visual_feat, acoustic_feat, textual_feat, W_v, W_a, W_t):
    uid = jnp.concatenate([user_id_emb, item_id_emb], axis=0)
    vis = jnp.concatenate([user_visual_emb, visual_feat], axis=0)
    aco = jnp.concatenate([user_acoustic_emb, acoustic_feat], axis=0)
    tex = jnp.concatenate([user_textual_emb, textual_feat], axis=0)

    sv = _spmm(adj_indices, adj_values, vis)
    sa = _spmm(adj_indices, adj_values, aco)
    st = _spmm(adj_indices, adj_values, tex)
    vis, aco, tex = _dense3(sv, sa, st, W_v[0], W_a[0], W_t[0])

    sv = _spmm(adj_indices, adj_values, vis)
    sa = _spmm(adj_indices, adj_values, aco)
    st = _spmm(adj_indices, adj_values, tex)
    fused = _dense3_fuse(sv, sa, st, W_v[1], W_a[1], W_t[1], uid)

    return (fused[:N_USERS], fused[N_USERS:])


# speedup vs baseline: 2.3282x; 2.3282x over previous
import functools

import jax
import jax.numpy as jnp
from jax import lax
from jax.experimental import pallas as pl
from jax.experimental.pallas import tpu as pltpu
from jax.experimental.pallas import tpu_sc as plsc

N_USERS = 20000
N_ITEMS = 30000
N_NODES = N_USERS + N_ITEMS
EMB = 64
N_EDGES = 800000

NUM_CORES = 2
NUM_SUBCORES = 16
ES = N_EDGES // NUM_SUBCORES
BLK = 2000
NBLK = ES // BLK
VPB = BLK // 16
CR = 6272
CPC = 4
N_PAD = NUM_CORES * CPC * CR
PT = CR // NUM_SUBCORES
GB = 128
CAP = 2048

ROW_BLK = 2000


def _sc_spmm_body(row_h, col_h, val_h, xv_h, xa_h, xt_h, ov_h, oa_h, ot_h,
                  rowb, colb, valb, cidx, crow1, cval, crow2,
                  xbv, xba, xbt, zbuf, sem, accv, acca, acct):
    c = lax.axis_index("c")
    s = lax.axis_index("s")

    zeros_f = jnp.zeros((16,), jnp.float32)
    zeros_i = jnp.zeros((16,), jnp.int32)

    def zb(j, _):
        for k in range(4):
            zbuf[j, pl.ds(k * 16, 16)] = zeros_f
        return 0
    lax.fori_loop(0, GB, zb, 0)

    accs = (accv, acca, acct)
    outs = (ov_h, oa_h, ot_h)
    xs = (xv_h, xa_h, xt_h)
    xbs = (xbv, xba, xbt)

    for ck in range(CPC):
        lo = (c * CPC + ck) * CR
        tb = s * PT
        for acc in accs:
            sync = pltpu.sync_copy
            sync(zbuf, acc.at[pl.ds(tb, GB)])
            sync(zbuf, acc.at[pl.ds(tb + GB, GB)])
            sync(zbuf, acc.at[pl.ds(tb + 2 * GB, GB)])
            sync(zbuf.at[pl.ds(0, PT - 3 * GB)], acc.at[pl.ds(tb + 3 * GB, PT - 3 * GB)])
        plsc.subcore_barrier()

        def blk_body(b, _):
            ebase = s * ES + b * BLK
            pltpu.sync_copy(row_h.at[pl.ds(ebase, BLK)], rowb)
            pltpu.sync_copy(col_h.at[pl.ds(ebase, BLK)], colb)
            pltpu.sync_copy(val_h.at[pl.ds(ebase, BLK)], valb)

            lane = lax.iota(jnp.int32, 16)

            def filt(v, cnt):
                rv = rowb[pl.ds(v * 16, 16)]
                m = (rv >= lo) & (rv < lo + CR)
                mi = m.astype(jnp.int32)
                incl = plsc.cumsum(mi)
                pos = jnp.where(m, cnt + incl - 1, jnp.int32(CAP) + lane)
                plsc.store_scatter(crow1, [pos], rv - lo)
                plsc.store_scatter(cidx, [pos], colb[pl.ds(v * 16, 16)])
                plsc.store_scatter(cval, [pos], valb[pl.ds(v * 16, 16)])
                return cnt + jnp.sum(mi)
            cnt = lax.fori_loop(0, VPB, filt, jnp.int32(0))
            cnt_pad = ((cnt + GB - 1) // GB) * GB

            def padw(w, _):
                p = cnt + w * 16

                @pl.when(p < cnt_pad)
                def _():
                    crow1[pl.ds(p, 16)] = zeros_i
                    cidx[pl.ds(p, 16)] = zeros_i
                    cval[pl.ds(p, 16)] = zeros_f
                return 0
            lax.fori_loop(0, GB // 16, padw, 0)

            nb = cnt_pad // GB

            def cp(bb, _):
                for k in range(8):
                    crow2[bb, pl.ds(k * 16, 16)] = crow1[pl.ds(bb * GB + k * 16, 16)]
                return 0
            lax.fori_loop(0, nb, cp, 0)

            def sub(bb, _):
                idxsl = cidx.at[pl.ds(bb * GB, GB)]
                pltpu.async_copy(xs[0].at[idxsl], xbv, sem).wait()
                pltpu.async_copy(xs[1].at[idxsl], xba, sem).wait()
                pltpu.async_copy(xs[2].at[idxsl], xbt, sem).wait()

                def rowf(j, _):
                    w = plsc.load_gather(
                        cval, [jnp.full((16,), bb * GB + j, jnp.int32)])
                    for xb in xbs:
                        for k in range(4):
                            xb[j, pl.ds(k * 16, 16)] = xb[j, pl.ds(k * 16, 16)] * w
                    return 0
                lax.fori_loop(0, GB, rowf, 0)

                rsl = crow2.at[bb]
                pltpu.sync_copy(xbv, accv.at[rsl], add=True)
                pltpu.sync_copy(xba, acca.at[rsl], add=True)
                pltpu.sync_copy(xbt, acct.at[rsl], add=True)
                return 0
            lax.fori_loop(0, nb, sub, 0)
            return 0
        lax.fori_loop(0, NBLK, blk_body, 0)
        plsc.subcore_barrier()

        for acc, out in zip(accs, outs):
            sync = pltpu.sync_copy
            sync(acc.at[pl.ds(tb, GB)], out.at[pl.ds(lo + tb, GB)])
            sync(acc.at[pl.ds(tb + GB, GB)], out.at[pl.ds(lo + tb + GB, GB)])
            sync(acc.at[pl.ds(tb + 2 * GB, GB)], out.at[pl.ds(lo + tb + 2 * GB, GB)])
            sync(acc.at[pl.ds(tb + 3 * GB, PT - 3 * GB)],
                 out.at[pl.ds(lo + tb + 3 * GB, PT - 3 * GB)])
        plsc.subcore_barrier()


@functools.partial(jax.jit, donate_argnums=())
def _sc_spmm(row, col, val, xv, xa, xt):
    out_t = [jax.ShapeDtypeStruct((N_PAD, EMB), jnp.float32)] * 3
    mesh = plsc.VectorSubcoreMesh(core_axis_name="c", subcore_axis_name="s")
    f = pl.kernel(
        _sc_spmm_body,
        out_type=out_t,
        mesh=mesh,
        compiler_params=pltpu.CompilerParams(needs_layout_passes=False, use_tc_tiling_on_sc=False),
        scratch_types=[
            pltpu.VMEM((BLK,), jnp.int32),
            pltpu.VMEM((BLK,), jnp.int32),
            pltpu.VMEM((BLK,), jnp.float32),
            pltpu.VMEM((CAP + 16,), jnp.int32),
            pltpu.VMEM((CAP + 16,), jnp.int32),
            pltpu.VMEM((CAP + 16,), jnp.float32),
            pltpu.VMEM((CAP // GB, GB), jnp.int32),
            pltpu.VMEM((GB, EMB), jnp.float32),
            pltpu.VMEM((GB, EMB), jnp.float32),
            pltpu.VMEM((GB, EMB), jnp.float32),
            pltpu.VMEM((GB, EMB), jnp.float32),
            pltpu.SemaphoreType.DMA,
            pltpu.VMEM_SHARED((CR, EMB), jnp.float32),
            pltpu.VMEM_SHARED((CR, EMB), jnp.float32),
            pltpu.VMEM_SHARED((CR, EMB), jnp.float32),
        ],
    )
    return f(row, col, val, xv, xa, xt)


def _dense3_body(sv, sa, st, wv, wa, wt, ov, oa, ot):
    for sref, wref, oref in ((sv, wv, ov), (sa, wa, oa), (st, wt, ot)):
        oref[...] = jax.nn.leaky_relu(
            lax.dot_general(sref[...], wref[...], (((1,), (1,)), ((), ())),
                            preferred_element_type=jnp.float32), 0.2)


def _dense3_fuse_body(sv, sa, st, wv, wa, wt, uid, out):
    acc = uid[...]
    for sref, wref in ((sv, wv), (sa, wa), (st, wt)):
        acc = acc + jax.nn.leaky_relu(
            lax.dot_general(sref[...], wref[...], (((1,), (1,)), ((), ())),
                            preferred_element_type=jnp.float32), 0.2)
    out[...] = acc


def _dense3(sv, sa, st, wv, wa, wt):
    grid = N_NODES // ROW_BLK
    row_spec = pl.BlockSpec((ROW_BLK, EMB), lambda i: (i, 0))
    w_spec = pl.BlockSpec((EMB, EMB), lambda i: (0, 0))
    return pl.pallas_call(
        _dense3_body,
        grid=(grid,),
        in_specs=[row_spec, row_spec, row_spec, w_spec, w_spec, w_spec],
        out_specs=[row_spec, row_spec, row_spec],
        out_shape=[jax.ShapeDtypeStruct((N_NODES, EMB), jnp.float32)] * 3,
    )(sv, sa, st, wv, wa, wt)


def _dense3_fuse(sv, sa, st, wv, wa, wt, uid):
    grid = N_NODES // ROW_BLK
    row_spec = pl.BlockSpec((ROW_BLK, EMB), lambda i: (i, 0))
    w_spec = pl.BlockSpec((EMB, EMB), lambda i: (0, 0))
    return pl.pallas_call(
        _dense3_fuse_body,
        grid=(grid,),
        in_specs=[row_spec, row_spec, row_spec, w_spec, w_spec, w_spec, row_spec],
        out_specs=row_spec,
        out_shape=jax.ShapeDtypeStruct((N_NODES, EMB), jnp.float32),
    )(sv, sa, st, wv, wa, wt, uid)


def kernel(adj_indices, adj_values, user_id_emb, item_id_emb,
           user_visual_emb, user_acoustic_emb, user_textual_emb,
           visual_feat, acoustic_feat, textual_feat, W_v, W_a, W_t):
    row = adj_indices[0]
    col = adj_indices[1]
    uid = jnp.concatenate([user_id_emb, item_id_emb], axis=0)
    vis = jnp.concatenate([user_visual_emb, visual_feat], axis=0)
    aco = jnp.concatenate([user_acoustic_emb, acoustic_feat], axis=0)
    tex = jnp.concatenate([user_textual_emb, textual_feat], axis=0)

    sv, sa, st = _sc_spmm(row, col, adj_values, vis, aco, tex)
    vis, aco, tex = _dense3(sv, sa, st, W_v[0], W_a[0], W_t[0])
    sv, sa, st = _sc_spmm(row, col, adj_values, vis, aco, tex)
    fused = _dense3_fuse(sv, sa, st, W_v[1], W_a[1], W_t[1], uid)

    return (fused[:N_USERS], fused[N_USERS:])


# speedup vs baseline: 2.6946x; 1.1574x over previous
import functools

import jax
import jax.numpy as jnp
from jax import lax
from jax.experimental import pallas as pl
from jax.experimental.pallas import tpu as pltpu
from jax.experimental.pallas import tpu_sc as plsc

N_USERS = 20000
N_ITEMS = 30000
N_NODES = N_USERS + N_ITEMS
EMB = 64
N_EDGES = 800000

NUM_CORES = 2
NUM_SUBCORES = 16
ES = N_EDGES // NUM_SUBCORES
BLK = 2000
NBLK = ES // BLK
VPB = BLK // 16
CR = 6272
CPC = 4
N_PAD = NUM_CORES * CPC * CR
PT = CR // NUM_SUBCORES
GB = 128
CAP = 2048

ROW_BLK = 2000


def _sc_spmm_body(row_h, col_h, val_h, xv_h, xa_h, xt_h, ov_h, oa_h, ot_h,
                  rowb, colb, valb, cidx, crow1, cval, crow2,
                  xbv, xba, xbt, zbuf, seme, semv, sema, semt,
                  accv, acca, acct):
    c = lax.axis_index("c")
    s = lax.axis_index("s")

    zeros_f = jnp.zeros((16,), jnp.float32)
    zeros_i = jnp.zeros((16,), jnp.int32)

    def zb(j, _):
        for k in range(4):
            zbuf[j, pl.ds(k * 16, 16)] = zeros_f
        return 0
    lax.fori_loop(0, GB, zb, 0)

    accs = (accv, acca, acct)
    outs = (ov_h, oa_h, ot_h)
    xs = (xv_h, xa_h, xt_h)
    xbs = (xbv, xba, xbt)

    for ck in range(CPC):
        lo = (c * CPC + ck) * CR
        tb = s * PT
        for acc in accs:
            sync = pltpu.sync_copy
            sync(zbuf, acc.at[pl.ds(tb, GB)])
            sync(zbuf, acc.at[pl.ds(tb + GB, GB)])
            sync(zbuf, acc.at[pl.ds(tb + 2 * GB, GB)])
            sync(zbuf.at[pl.ds(0, PT - 3 * GB)], acc.at[pl.ds(tb + 3 * GB, PT - 3 * GB)])
        plsc.subcore_barrier()

        def blk_body(b, _):
            ebase = s * ES + b * BLK
            d1 = pltpu.async_copy(row_h.at[pl.ds(ebase, BLK)], rowb, seme)
            d2 = pltpu.async_copy(col_h.at[pl.ds(ebase, BLK)], colb, seme)
            d3 = pltpu.async_copy(val_h.at[pl.ds(ebase, BLK)], valb, seme)
            d1.wait()
            d2.wait()
            d3.wait()

            lane = lax.iota(jnp.int32, 16)

            def filt(v, cnt):
                rv = rowb[pl.ds(v * 16, 16)]
                m = (rv >= lo) & (rv < lo + CR)
                mi = m.astype(jnp.int32)
                incl = plsc.cumsum(mi)
                pos = jnp.where(m, cnt + incl - 1, jnp.int32(CAP) + lane)
                plsc.store_scatter(crow1, [pos], rv - lo)
                plsc.store_scatter(cidx, [pos], colb[pl.ds(v * 16, 16)])
                plsc.store_scatter(cval, [pos], valb[pl.ds(v * 16, 16)])
                return cnt + jnp.sum(mi)
            cnt = lax.fori_loop(0, VPB, filt, jnp.int32(0))
            cnt_pad = ((cnt + GB - 1) // GB) * GB

            def padw(w, _):
                p = cnt + w * 16

                @pl.when(p < cnt_pad)
                def _():
                    crow1[pl.ds(p, 16)] = zeros_i
                    cidx[pl.ds(p, 16)] = zeros_i
                    cval[pl.ds(p, 16)] = zeros_f
                return 0
            lax.fori_loop(0, GB // 16, padw, 0)

            nb = cnt_pad // GB

            def cp(bb, _):
                for k in range(8):
                    crow2[bb, pl.ds(k * 16, 16)] = crow1[pl.ds(bb * GB + k * 16, 16)]
                return 0
            lax.fori_loop(0, nb, cp, 0)

            def sub(bb, _):
                base = bb * GB
                idxsl = cidx.at[pl.ds(base, GB)]
                dv = pltpu.async_copy(xs[0].at[idxsl], xbv, semv)
                da = pltpu.async_copy(xs[1].at[idxsl], xba, sema)
                dt = pltpu.async_copy(xs[2].at[idxsl], xbt, semt)
                rsl = crow2.at[bb]
                for d, xb, acc in ((dv, xbv, accv), (da, xba, acca), (dt, xbt, acct)):
                    d.wait()

                    def grp(g, _, xb=xb):
                        for j in range(16):
                            w = plsc.load_gather(
                                cval, [jnp.full((16,), base + g * 16 + j, jnp.int32)])
                            r = g * 16 + j
                            for k in range(4):
                                xb[r, pl.ds(k * 16, 16)] = (
                                    xb[r, pl.ds(k * 16, 16)] * w)
                        return 0
                    lax.fori_loop(0, GB // 16, grp, 0)
                    pltpu.sync_copy(xb, acc.at[rsl], add=True)
                return 0
            lax.fori_loop(0, nb, sub, 0)
            return 0
        lax.fori_loop(0, NBLK, blk_body, 0)
        plsc.subcore_barrier()

        for acc, out in zip(accs, outs):
            sync = pltpu.sync_copy
            sync(acc.at[pl.ds(tb, GB)], out.at[pl.ds(lo + tb, GB)])
            sync(acc.at[pl.ds(tb + GB, GB)], out.at[pl.ds(lo + tb + GB, GB)])
            sync(acc.at[pl.ds(tb + 2 * GB, GB)], out.at[pl.ds(lo + tb + 2 * GB, GB)])
            sync(acc.at[pl.ds(tb + 3 * GB, PT - 3 * GB)],
                 out.at[pl.ds(lo + tb + 3 * GB, PT - 3 * GB)])
        plsc.subcore_barrier()


@functools.partial(jax.jit, donate_argnums=())
def _sc_spmm(row, col, val, xv, xa, xt):
    out_t = [jax.ShapeDtypeStruct((N_PAD, EMB), jnp.float32)] * 3
    mesh = plsc.VectorSubcoreMesh(core_axis_name="c", subcore_axis_name="s")
    f = pl.kernel(
        _sc_spmm_body,
        out_type=out_t,
        mesh=mesh,
        compiler_params=pltpu.CompilerParams(needs_layout_passes=False, use_tc_tiling_on_sc=False),
        scratch_types=[
            pltpu.VMEM((BLK,), jnp.int32),
            pltpu.VMEM((BLK,), jnp.int32),
            pltpu.VMEM((BLK,), jnp.float32),
            pltpu.VMEM((CAP + 16,), jnp.int32),
            pltpu.VMEM((CAP + 16,), jnp.int32),
            pltpu.VMEM((CAP + 16,), jnp.float32),
            pltpu.VMEM((CAP // GB, GB), jnp.int32),
            pltpu.VMEM((GB, EMB), jnp.float32),
            pltpu.VMEM((GB, EMB), jnp.float32),
            pltpu.VMEM((GB, EMB), jnp.float32),
            pltpu.VMEM((GB, EMB), jnp.float32),
            pltpu.SemaphoreType.DMA,
            pltpu.SemaphoreType.DMA,
            pltpu.SemaphoreType.DMA,
            pltpu.SemaphoreType.DMA,
            pltpu.VMEM_SHARED((CR, EMB), jnp.float32),
            pltpu.VMEM_SHARED((CR, EMB), jnp.float32),
            pltpu.VMEM_SHARED((CR, EMB), jnp.float32),
        ],
    )
    return f(row, col, val, xv, xa, xt)


def _dense3_body(sv, sa, st, wv, wa, wt, ov, oa, ot):
    for sref, wref, oref in ((sv, wv, ov), (sa, wa, oa), (st, wt, ot)):
        oref[...] = jax.nn.leaky_relu(
            lax.dot_general(sref[...], wref[...], (((1,), (1,)), ((), ())),
                            preferred_element_type=jnp.float32), 0.2)


def _dense3_fuse_body(sv, sa, st, wv, wa, wt, uid, out):
    acc = uid[...]
    for sref, wref in ((sv, wv), (sa, wa), (st, wt)):
        acc = acc + jax.nn.leaky_relu(
            lax.dot_general(sref[...], wref[...], (((1,), (1,)), ((), ())),
                            preferred_element_type=jnp.float32), 0.2)
    out[...] = acc


def _dense3(sv, sa, st, wv, wa, wt):
    grid = N_NODES // ROW_BLK
    row_spec = pl.BlockSpec((ROW_BLK, EMB), lambda i: (i, 0))
    w_spec = pl.BlockSpec((EMB, EMB), lambda i: (0, 0))
    return pl.pallas_call(
        _dense3_body,
        grid=(grid,),
        in_specs=[row_spec, row_spec, row_spec, w_spec, w_spec, w_spec],
        out_specs=[row_spec, row_spec, row_spec],
        out_shape=[jax.ShapeDtypeStruct((N_NODES, EMB), jnp.float32)] * 3,
    )(sv, sa, st, wv, wa, wt)


def _dense3_fuse(sv, sa, st, wv, wa, wt, uid):
    grid = N_NODES // ROW_BLK
    row_spec = pl.BlockSpec((ROW_BLK, EMB), lambda i: (i, 0))
    w_spec = pl.BlockSpec((EMB, EMB), lambda i: (0, 0))
    return pl.pallas_call(
        _dense3_fuse_body,
        grid=(grid,),
        in_specs=[row_spec, row_spec, row_spec, w_spec, w_spec, w_spec, row_spec],
        out_specs=row_spec,
        out_shape=jax.ShapeDtypeStruct((N_NODES, EMB), jnp.float32),
    )(sv, sa, st, wv, wa, wt, uid)


def kernel(adj_indices, adj_values, user_id_emb, item_id_emb,
           user_visual_emb, user_acoustic_emb, user_textual_emb,
           visual_feat, acoustic_feat, textual_feat, W_v, W_a, W_t):
    row = adj_indices[0]
    col = adj_indices[1]
    uid = jnp.concatenate([user_id_emb, item_id_emb], axis=0)
    vis = jnp.concatenate([user_visual_emb, visual_feat], axis=0)
    aco = jnp.concatenate([user_acoustic_emb, acoustic_feat], axis=0)
    tex = jnp.concatenate([user_textual_emb, textual_feat], axis=0)

    sv, sa, st = _sc_spmm(row, col, adj_values, vis, aco, tex)
    vis, aco, tex = _dense3(sv, sa, st, W_v[0], W_a[0], W_t[0])
    sv, sa, st = _sc_spmm(row, col, adj_values, vis, aco, tex)
    fused = _dense3_fuse(sv, sa, st, W_v[1], W_a[1], W_t[1], uid)

    return (fused[:N_USERS], fused[N_USERS:])


# speedup vs baseline: 4.1597x; 1.5437x over previous
import functools

import jax
import jax.numpy as jnp
from jax import lax
from jax.experimental import pallas as pl
from jax.experimental.pallas import tpu as pltpu
from jax.experimental.pallas import tpu_sc as plsc

N_USERS = 20000
N_ITEMS = 30000
N_NODES = N_USERS + N_ITEMS
EMB = 64
N_EDGES = 800000

NUM_CORES = 2
NUM_SUBCORES = 16
ES = N_EDGES // NUM_SUBCORES
BLK = 2000
NBLK = ES // BLK
VPB = BLK // 16
CR = 6272
CPC = 4
N_PAD = NUM_CORES * CPC * CR
PT = CR // NUM_SUBCORES
GB = 64
CAP = 2048

ROW_BLK = 2000


def _sc_spmm_body(row_h, col_h, val_h, xv_h, xa_h, xt_h, ov_h, oa_h, ot_h,
                  rowb, colb, valb, cidx, cval, crow2,
                  xbv, xba, xbt, xcv, xca, xct, zbuf,
                  seme, semv, sema, semt, semv2, sema2, semt2,
                  accv, acca, acct):
    c = lax.axis_index("c")
    s = lax.axis_index("s")

    zeros_f = jnp.zeros((16,), jnp.float32)
    zeros_i = jnp.zeros((16,), jnp.int32)

    def zb(j, _):
        for k in range(4):
            zbuf[j, pl.ds(k * 16, 16)] = zeros_f
        return 0
    lax.fori_loop(0, GB, zb, 0)

    accs = (accv, acca, acct)
    outs = (ov_h, oa_h, ot_h)
    xs = (xv_h, xa_h, xt_h)
    xbs = (xbv, xba, xbt)

    def chunk_body(ck, _):
        lo = (c * CPC + ck) * CR
        tb = s * PT
        for acc in accs:
            sync = pltpu.sync_copy
            for q in range(PT // GB):
                sync(zbuf, acc.at[pl.ds(tb + q * GB, GB)])
            sync(zbuf.at[pl.ds(0, PT % GB)],
                 acc.at[pl.ds(tb + (PT // GB) * GB, PT % GB)])
        plsc.subcore_barrier()

        def blk_body(b, _):
            ebase = s * ES + b * BLK
            d1 = pltpu.async_copy(row_h.at[pl.ds(ebase, BLK)], rowb, seme)
            d2 = pltpu.async_copy(col_h.at[pl.ds(ebase, BLK)], colb, seme)
            d3 = pltpu.async_copy(val_h.at[pl.ds(ebase, BLK)], valb, seme)
            d1.wait()
            d2.wait()
            d3.wait()

            lane = lax.iota(jnp.int32, 16)

            def filt(v, cnt):
                rv = rowb[pl.ds(v * 16, 16)]
                m = (rv >= lo) & (rv < lo + CR)
                mi = m.astype(jnp.int32)
                incl = plsc.cumsum(mi)
                pos = jnp.where(m, cnt + incl - 1, jnp.int32(CAP) + lane)
                plsc.store_scatter(crow2, [pos // GB, pos % GB], rv - lo)
                plsc.store_scatter(cidx, [pos], colb[pl.ds(v * 16, 16)])
                plsc.store_scatter(cval, [pos], valb[pl.ds(v * 16, 16)])
                return cnt + jnp.sum(mi)
            cnt = lax.fori_loop(0, VPB, filt, jnp.int32(0))
            cnt_pad = ((cnt + GB - 1) // GB) * GB

            def padw(w, _):
                p = cnt + w * 16

                @pl.when(p < cnt_pad)
                def _():
                    crow2[p // GB, pl.ds(p % GB, 16)] = zeros_i
                    cidx[pl.ds(p, 16)] = zeros_i
                    cval[pl.ds(p, 16)] = zeros_f
                return 0
            lax.fori_loop(0, GB // 16, padw, 0)

            nb = cnt_pad // GB

            bufsets = ((xbv, xba, xbt), (xcv, xca, xct))
            semsets = ((semv, sema, semt), (semv2, sema2, semt2))

            def fire(bb, si):
                idxsl = cidx.at[pl.ds(bb * GB, GB)]
                for i in range(3):
                    pltpu.async_copy(xs[i].at[idxsl], bufsets[si][i], semsets[si][i])

            def scale_scatter(bb, si):
                base = bb * GB
                idxsl = cidx.at[pl.ds(base, GB)]
                rsl = crow2.at[bb]
                for i in range(3):
                    xb = bufsets[si][i]
                    pltpu.make_async_copy(
                        xs[i].at[idxsl], xb, semsets[si][i]).wait()

                    def grp(g, _, xb=xb):
                        for j in range(4):
                            w = plsc.load_gather(
                                cval, [jnp.full((16,), base + g * 4 + j, jnp.int32)])
                            r = g * 4 + j
                            for k in range(4):
                                xb[r, pl.ds(k * 16, 16)] = (
                                    xb[r, pl.ds(k * 16, 16)] * w)
                        return 0
                    lax.fori_loop(0, GB // 4, grp, 0)
                    pltpu.sync_copy(xb, accs[i].at[rsl], add=True)

            @pl.when(nb > 0)
            def _():
                fire(0, 0)

            def pair(k, _):
                b0 = 2 * k
                b1 = b0 + 1

                @pl.when(b1 < nb)
                def _():
                    fire(b1, 1)
                scale_scatter(b0, 0)

                @pl.when(b1 < nb)
                def _():
                    @pl.when(b0 + 2 < nb)
                    def _():
                        fire(b0 + 2, 0)
                    scale_scatter(b1, 1)
                return 0
            lax.fori_loop(0, (nb + 1) // 2, pair, 0)
            return 0
        lax.fori_loop(0, NBLK, blk_body, 0)
        plsc.subcore_barrier()

        for acc, out in zip(accs, outs):
            sync = pltpu.sync_copy
            for q in range(PT // GB):
                sync(acc.at[pl.ds(tb + q * GB, GB)], out.at[pl.ds(lo + tb + q * GB, GB)])
            sync(acc.at[pl.ds(tb + (PT // GB) * GB, PT % GB)],
                 out.at[pl.ds(lo + tb + (PT // GB) * GB, PT % GB)])
        plsc.subcore_barrier()
        return 0
    lax.fori_loop(0, CPC, chunk_body, 0)


@functools.partial(jax.jit, donate_argnums=())
def _sc_spmm(row, col, val, xv, xa, xt):
    out_t = [jax.ShapeDtypeStruct((N_PAD, EMB), jnp.float32)] * 3
    mesh = plsc.VectorSubcoreMesh(core_axis_name="c", subcore_axis_name="s")
    f = pl.kernel(
        _sc_spmm_body,
        out_type=out_t,
        mesh=mesh,
        compiler_params=pltpu.CompilerParams(needs_layout_passes=False, use_tc_tiling_on_sc=False),
        scratch_types=[
            pltpu.VMEM((BLK,), jnp.int32),
            pltpu.VMEM((BLK,), jnp.int32),
            pltpu.VMEM((BLK,), jnp.float32),
            pltpu.VMEM((CAP + 16,), jnp.int32),
            pltpu.VMEM((CAP + 16,), jnp.float32),
            pltpu.VMEM((CAP // GB + 1, GB), jnp.int32),
            pltpu.VMEM((GB, EMB), jnp.float32),
            pltpu.VMEM((GB, EMB), jnp.float32),
            pltpu.VMEM((GB, EMB), jnp.float32),
            pltpu.VMEM((GB, EMB), jnp.float32),
            pltpu.VMEM((GB, EMB), jnp.float32),
            pltpu.VMEM((GB, EMB), jnp.float32),
            pltpu.VMEM((GB, EMB), jnp.float32),
            pltpu.SemaphoreType.DMA,
            pltpu.SemaphoreType.DMA,
            pltpu.SemaphoreType.DMA,
            pltpu.SemaphoreType.DMA,
            pltpu.SemaphoreType.DMA,
            pltpu.SemaphoreType.DMA,
            pltpu.SemaphoreType.DMA,
            pltpu.VMEM_SHARED((CR, EMB), jnp.float32),
            pltpu.VMEM_SHARED((CR, EMB), jnp.float32),
            pltpu.VMEM_SHARED((CR, EMB), jnp.float32),
        ],
    )
    return f(row, col, val, xv, xa, xt)


def _dense3_body(sv, sa, st, wv, wa, wt, ov, oa, ot):
    for sref, wref, oref in ((sv, wv, ov), (sa, wa, oa), (st, wt, ot)):
        oref[...] = jax.nn.leaky_relu(
            lax.dot_general(sref[...], wref[...], (((1,), (1,)), ((), ())),
                            preferred_element_type=jnp.float32), 0.2)


def _dense3_fuse_body(sv, sa, st, wv, wa, wt, uid, out):
    acc = uid[...]
    for sref, wref in ((sv, wv), (sa, wa), (st, wt)):
        acc = acc + jax.nn.leaky_relu(
            lax.dot_general(sref[...], wref[...], (((1,), (1,)), ((), ())),
                            preferred_element_type=jnp.float32), 0.2)
    out[...] = acc


def _dense3(sv, sa, st, wv, wa, wt):
    grid = N_NODES // ROW_BLK
    row_spec = pl.BlockSpec((ROW_BLK, EMB), lambda i: (i, 0))
    w_spec = pl.BlockSpec((EMB, EMB), lambda i: (0, 0))
    return pl.pallas_call(
        _dense3_body,
        grid=(grid,),
        in_specs=[row_spec, row_spec, row_spec, w_spec, w_spec, w_spec],
        out_specs=[row_spec, row_spec, row_spec],
        out_shape=[jax.ShapeDtypeStruct((N_NODES, EMB), jnp.float32)] * 3,
    )(sv, sa, st, wv, wa, wt)


def _dense3_fuse(sv, sa, st, wv, wa, wt, uid):
    grid = N_NODES // ROW_BLK
    row_spec = pl.BlockSpec((ROW_BLK, EMB), lambda i: (i, 0))
    w_spec = pl.BlockSpec((EMB, EMB), lambda i: (0, 0))
    return pl.pallas_call(
        _dense3_fuse_body,
        grid=(grid,),
        in_specs=[row_spec, row_spec, row_spec, w_spec, w_spec, w_spec, row_spec],
        out_specs=row_spec,
        out_shape=jax.ShapeDtypeStruct((N_NODES, EMB), jnp.float32),
    )(sv, sa, st, wv, wa, wt, uid)


def kernel(adj_indices, adj_values, user_id_emb, item_id_emb,
           user_visual_emb, user_acoustic_emb, user_textual_emb,
           visual_feat, acoustic_feat, textual_feat, W_v, W_a, W_t):
    row = adj_indices[0]
    col = adj_indices[1]
    uid = jnp.concatenate([user_id_emb, item_id_emb], axis=0)
    vis = jnp.concatenate([user_visual_emb, visual_feat], axis=0)
    aco = jnp.concatenate([user_acoustic_emb, acoustic_feat], axis=0)
    tex = jnp.concatenate([user_textual_emb, textual_feat], axis=0)

    sv, sa, st = _sc_spmm(row, col, adj_values, vis, aco, tex)
    vis, aco, tex = _dense3(sv, sa, st, W_v[0], W_a[0], W_t[0])
    sv, sa, st = _sc_spmm(row, col, adj_values, vis, aco, tex)
    fused = _dense3_fuse(sv, sa, st, W_v[1], W_a[1], W_t[1], uid)

    return (fused[:N_USERS], fused[N_USERS:])


# speedup vs baseline: 4.1965x; 1.0088x over previous
import functools

import jax
import jax.numpy as jnp
from jax import lax
from jax.experimental import pallas as pl
from jax.experimental.pallas import tpu as pltpu
from jax.experimental.pallas import tpu_sc as plsc

N_USERS = 20000
N_ITEMS = 30000
N_NODES = N_USERS + N_ITEMS
EMB = 64
N_EDGES = 800000

NUM_CORES = 2
NUM_SUBCORES = 16
ES = N_EDGES // NUM_SUBCORES
BLK = 2000
NBLK = ES // BLK
VPB = BLK // 16
CR = 6272
CPC = 4
N_PAD = NUM_CORES * CPC * CR
PT = CR // NUM_SUBCORES
GB = 64
CAP = 2048

ROW_BLK = 2000


def _sc_spmm_body(row_h, col_h, val_h, xv_h, xa_h, xt_h, ov_h, oa_h, ot_h,
                  rowb, colb, valb, cidx, cval, crow2,
                  xbv, xba, xbt, xcv, xca, xct, zbuf,
                  seme, semv, sema, semt, semv2, sema2, semt2,
                  semsc0, semsc1, accv, acca, acct):
    c = lax.axis_index("c")
    s = lax.axis_index("s")

    zeros_f = jnp.zeros((16,), jnp.float32)
    zeros_i = jnp.zeros((16,), jnp.int32)

    def zb(j, _):
        for k in range(4):
            zbuf[j, pl.ds(k * 16, 16)] = zeros_f
        return 0
    lax.fori_loop(0, GB, zb, 0)

    accs = (accv, acca, acct)
    outs = (ov_h, oa_h, ot_h)
    xs = (xv_h, xa_h, xt_h)
    xbs = (xbv, xba, xbt)

    def chunk_body(ck, _):
        lo = (c * CPC + ck) * CR
        tb = s * PT
        for acc in accs:
            sync = pltpu.sync_copy
            for q in range(PT // GB):
                sync(zbuf, acc.at[pl.ds(tb + q * GB, GB)])
            sync(zbuf.at[pl.ds(0, PT % GB)],
                 acc.at[pl.ds(tb + (PT // GB) * GB, PT % GB)])
        plsc.subcore_barrier()

        def blk_body(b, _):
            ebase = s * ES + b * BLK
            d1 = pltpu.async_copy(row_h.at[pl.ds(ebase, BLK)], rowb, seme)
            d2 = pltpu.async_copy(col_h.at[pl.ds(ebase, BLK)], colb, seme)
            d3 = pltpu.async_copy(val_h.at[pl.ds(ebase, BLK)], valb, seme)
            d1.wait()
            d2.wait()
            d3.wait()

            lane = lax.iota(jnp.int32, 16)

            def filt(v, cntv):
                rv = rowb[pl.ds(v * 16, 16)]
                m = (rv >= lo) & (rv < lo + CR)
                mi = m.astype(jnp.int32)
                incl = plsc.cumsum(mi)
                pos = jnp.where(m, cntv + incl - 1, jnp.int32(CAP) + lane)
                plsc.store_scatter(crow2, [pos // GB, pos % GB], rv - lo)
                plsc.store_scatter(cidx, [pos], colb[pl.ds(v * 16, 16)])
                plsc.store_scatter(cval, [pos], valb[pl.ds(v * 16, 16)])
                return cntv + jnp.sum(mi)
            cnt = lax.fori_loop(0, VPB, filt, jnp.int32(0))
            cnt_pad = ((cnt + GB - 1) // GB) * GB

            def padw(w, _):
                p = cnt + w * 16

                @pl.when(p < cnt_pad)
                def _():
                    crow2[p // GB, pl.ds(p % GB, 16)] = zeros_i
                    cidx[pl.ds(p, 16)] = zeros_i
                    cval[pl.ds(p, 16)] = zeros_f
                return 0
            lax.fori_loop(0, GB // 16, padw, 0)

            nb = cnt_pad // GB

            bufsets = ((xbv, xba, xbt), (xcv, xca, xct))
            semsets = ((semv, sema, semt), (semv2, sema2, semt2))
            semsc = (semsc0, semsc1)

            def fire(bb, si):
                idxsl = cidx.at[pl.ds(bb * GB, GB)]
                for i in range(3):
                    pltpu.async_copy(xs[i].at[idxsl], bufsets[si][i], semsets[si][i])

            def scale_scatter(bb, si):
                base = bb * GB
                idxsl = cidx.at[pl.ds(base, GB)]
                rsl = crow2.at[bb]
                for i in range(3):
                    xb = bufsets[si][i]
                    pltpu.make_async_copy(
                        xs[i].at[idxsl], xb, semsets[si][i]).wait()

                    def grp(g, _, xb=xb):
                        for j in range(4):
                            w = plsc.load_gather(
                                cval, [jnp.full((16,), base + g * 4 + j, jnp.int32)])
                            r = g * 4 + j
                            for k in range(4):
                                xb[r, pl.ds(k * 16, 16)] = (
                                    xb[r, pl.ds(k * 16, 16)] * w)
                        return 0
                    lax.fori_loop(0, GB // 4, grp, 0)
                    pltpu.async_copy(xb, accs[i].at[rsl], semsc[si], add=True)

            def waitsc(si):
                rsl0 = crow2.at[0]
                for i in range(3):
                    pltpu.make_async_copy(
                        bufsets[si][i], accs[i].at[rsl0], semsc[si]).wait()

            @pl.when(nb > 0)
            def _():
                fire(0, 0)

            def pair(k, _):
                b0 = 2 * k
                b1 = b0 + 1

                @pl.when(b1 < nb)
                def _():
                    @pl.when(k >= 1)
                    def _():
                        waitsc(1)
                    fire(b1, 1)
                scale_scatter(b0, 0)

                @pl.when(b1 < nb)
                def _():
                    @pl.when(b0 + 2 < nb)
                    def _():
                        waitsc(0)
                        fire(b0 + 2, 0)
                    scale_scatter(b1, 1)
                return 0
            lax.fori_loop(0, (nb + 1) // 2, pair, 0)

            @pl.when(nb >= 1)
            def _():
                waitsc(0)

            @pl.when(nb >= 2)
            def _():
                waitsc(1)
            return 0
        lax.fori_loop(0, NBLK, blk_body, 0)
        plsc.subcore_barrier()

        for acc, out in zip(accs, outs):
            sync = pltpu.sync_copy
            for q in range(PT // GB):
                sync(acc.at[pl.ds(tb + q * GB, GB)], out.at[pl.ds(lo + tb + q * GB, GB)])
            sync(acc.at[pl.ds(tb + (PT // GB) * GB, PT % GB)],
                 out.at[pl.ds(lo + tb + (PT // GB) * GB, PT % GB)])
        plsc.subcore_barrier()
        return 0
    lax.fori_loop(0, CPC, chunk_body, 0)


@functools.partial(jax.jit, donate_argnums=())
def _sc_spmm(row, col, val, xv, xa, xt):
    out_t = [jax.ShapeDtypeStruct((N_PAD, EMB), jnp.float32)] * 3
    mesh = plsc.VectorSubcoreMesh(core_axis_name="c", subcore_axis_name="s")
    f = pl.kernel(
        _sc_spmm_body,
        out_type=out_t,
        mesh=mesh,
        compiler_params=pltpu.CompilerParams(needs_layout_passes=False, use_tc_tiling_on_sc=False),
        scratch_types=[
            pltpu.VMEM((BLK,), jnp.int32),
            pltpu.VMEM((BLK,), jnp.int32),
            pltpu.VMEM((BLK,), jnp.float32),
            pltpu.VMEM((CAP + 16,), jnp.int32),
            pltpu.VMEM((CAP + 16,), jnp.float32),
            pltpu.VMEM((CAP // GB + 1, GB), jnp.int32),
            pltpu.VMEM((GB, EMB), jnp.float32),
            pltpu.VMEM((GB, EMB), jnp.float32),
            pltpu.VMEM((GB, EMB), jnp.float32),
            pltpu.VMEM((GB, EMB), jnp.float32),
            pltpu.VMEM((GB, EMB), jnp.float32),
            pltpu.VMEM((GB, EMB), jnp.float32),
            pltpu.VMEM((GB, EMB), jnp.float32),
            pltpu.SemaphoreType.DMA,
            pltpu.SemaphoreType.DMA,
            pltpu.SemaphoreType.DMA,
            pltpu.SemaphoreType.DMA,
            pltpu.SemaphoreType.DMA,
            pltpu.SemaphoreType.DMA,
            pltpu.SemaphoreType.DMA,
            pltpu.SemaphoreType.DMA,
            pltpu.SemaphoreType.DMA,
            pltpu.VMEM_SHARED((CR, EMB), jnp.float32),
            pltpu.VMEM_SHARED((CR, EMB), jnp.float32),
            pltpu.VMEM_SHARED((CR, EMB), jnp.float32),
        ],
    )
    return f(row, col, val, xv, xa, xt)


def _dense3_body(sv, sa, st, wv, wa, wt, ov, oa, ot):
    for sref, wref, oref in ((sv, wv, ov), (sa, wa, oa), (st, wt, ot)):
        oref[...] = jax.nn.leaky_relu(
            lax.dot_general(sref[...], wref[...], (((1,), (1,)), ((), ())),
                            preferred_element_type=jnp.float32), 0.2)


def _dense3_fuse_body(sv, sa, st, wv, wa, wt, uid, out):
    acc = uid[...]
    for sref, wref in ((sv, wv), (sa, wa), (st, wt)):
        acc = acc + jax.nn.leaky_relu(
            lax.dot_general(sref[...], wref[...], (((1,), (1,)), ((), ())),
                            preferred_element_type=jnp.float32), 0.2)
    out[...] = acc


def _dense3(sv, sa, st, wv, wa, wt):
    grid = N_NODES // ROW_BLK
    row_spec = pl.BlockSpec((ROW_BLK, EMB), lambda i: (i, 0))
    w_spec = pl.BlockSpec((EMB, EMB), lambda i: (0, 0))
    return pl.pallas_call(
        _dense3_body,
        grid=(grid,),
        in_specs=[row_spec, row_spec, row_spec, w_spec, w_spec, w_spec],
        out_specs=[row_spec, row_spec, row_spec],
        out_shape=[jax.ShapeDtypeStruct((N_NODES, EMB), jnp.float32)] * 3,
    )(sv, sa, st, wv, wa, wt)


def _dense3_fuse(sv, sa, st, wv, wa, wt, uid):
    grid = N_NODES // ROW_BLK
    row_spec = pl.BlockSpec((ROW_BLK, EMB), lambda i: (i, 0))
    w_spec = pl.BlockSpec((EMB, EMB), lambda i: (0, 0))
    return pl.pallas_call(
        _dense3_fuse_body,
        grid=(grid,),
        in_specs=[row_spec, row_spec, row_spec, w_spec, w_spec, w_spec, row_spec],
        out_specs=row_spec,
        out_shape=jax.ShapeDtypeStruct((N_NODES, EMB), jnp.float32),
    )(sv, sa, st, wv, wa, wt, uid)


def kernel(adj_indices, adj_values, user_id_emb, item_id_emb,
           user_visual_emb, user_acoustic_emb, user_textual_emb,
           visual_feat, acoustic_feat, textual_feat, W_v, W_a, W_t):
    row = adj_indices[0]
    col = adj_indices[1]
    uid = jnp.concatenate([user_id_emb, item_id_emb], axis=0)
    vis = jnp.concatenate([user_visual_emb, visual_feat], axis=0)
    aco = jnp.concatenate([user_acoustic_emb, acoustic_feat], axis=0)
    tex = jnp.concatenate([user_textual_emb, textual_feat], axis=0)

    sv, sa, st = _sc_spmm(row, col, adj_values, vis, aco, tex)
    vis, aco, tex = _dense3(sv, sa, st, W_v[0], W_a[0], W_t[0])
    sv, sa, st = _sc_spmm(row, col, adj_values, vis, aco, tex)
    fused = _dense3_fuse(sv, sa, st, W_v[1], W_a[1], W_t[1], uid)

    return (fused[:N_USERS], fused[N_USERS:])
